# baseline (speedup 1.0000x reference)
import functools

import jax
import jax.numpy as jnp
from jax import lax
from jax.experimental import pallas as pl
from jax.experimental.pallas import tpu as pltpu

_DECAY_MIN = 0.9
_DECAY_MAX = 0.999

_C = 512


def _bf(x):
    return x.astype(jnp.bfloat16)


def _body(scal_ref, x_ref, w_ref, wo_ref, bo_ref, y_ref, sfin_ref,
          mask_scr, dpow_scr, rpow_scr, *, nc, M):
    c = pl.program_id(1)
    bg = scal_ref[0]
    ld = scal_ref[1]
    decay_c = scal_ref[2]

    @pl.when(c == 0)
    def _init():
        ti = lax.broadcasted_iota(jnp.int32, (_C, _C), 0).astype(jnp.float32)
        si = lax.broadcasted_iota(jnp.int32, (_C, _C), 1).astype(jnp.float32)
        mask_scr[...] = _bf(jnp.where(si <= ti, jnp.exp((ti - si) * ld), 0.0))
        tc = lax.broadcasted_iota(jnp.int32, (_C, M), 0).astype(jnp.float32)
        dpow_scr[...] = jnp.exp((tc + 1.0) * ld)
        rpow_scr[...] = jnp.exp((_C - 1.0 - tc) * ld)
        sfin_ref[...] = jnp.zeros_like(sfin_ref)

    xb = _bf(x_ref[0])
    qkvg = jnp.dot(xb, w_ref[...], preferred_element_type=jnp.float32)
    q = qkvg[:, 0:M]
    k = qkvg[:, M:2 * M]
    v = qkvg[:, 2 * M:3 * M]
    g = jax.nn.sigmoid(qkvg[:, 3 * M:4 * M] + bg)
    q16 = _bf(q)
    gk = k * g
    gv16 = _bf(v * g)

    a = lax.dot_general(q16, _bf(gk), (((1,), (1,)), ((), ())),
                        preferred_element_type=jnp.float32)
    a16 = _bf(a) * mask_scr[...]
    intra = jnp.dot(a16, gv16, preferred_element_type=jnp.float32)

    s_prev = sfin_ref[0]
    inter = jnp.dot(_bf(q * dpow_scr[...]), _bf(s_prev),
                    preferred_element_type=jnp.float32)

    out = _bf(intra + inter)
    y_ref[0] = jnp.dot(out, wo_ref[...],
                       preferred_element_type=jnp.float32) + bo_ref[...]

    ktv = lax.dot_general(_bf(gk * rpow_scr[...]), gv16,
                          (((0,), (0,)), ((), ())),
                          preferred_element_type=jnp.float32)
    sfin_ref[0] = decay_c * s_prev + ktv


def kernel(x, Wq, Wk, Wv, Wo, bo, Wg, bg, decay_param):
    B, L, D = x.shape
    M = Wq.shape[1]
    nc = L // _C

    decay = _DECAY_MIN + jax.nn.sigmoid(decay_param[0]) * (_DECAY_MAX - _DECAY_MIN)
    ld = jnp.log(decay)
    scal = jnp.stack([bg[0], ld, decay ** _C]).astype(jnp.float32)

    w_all = jnp.concatenate(
        [Wq, Wk, Wv, jnp.tile(Wg, (1, M))], axis=1).astype(jnp.bfloat16)
    wo16 = Wo.astype(jnp.bfloat16)
    bo2 = bo.reshape(1, D)

    body = functools.partial(_body, nc=nc, M=M)
    y, s_final = pl.pallas_call(
        body,
        grid=(B, nc),
        in_specs=[
            pl.BlockSpec(memory_space=pltpu.SMEM),
            pl.BlockSpec((1, _C, D), lambda b, c: (b, c, 0)),
            pl.BlockSpec((D, 4 * M), lambda b, c: (0, 0)),
            pl.BlockSpec((M, D), lambda b, c: (0, 0)),
            pl.BlockSpec((1, D), lambda b, c: (0, 0)),
        ],
        out_specs=[
            pl.BlockSpec((1, _C, D), lambda b, c: (b, c, 0)),
            pl.BlockSpec((1, M, M), lambda b, c: (b, 0, 0)),
        ],
        out_shape=[
            jax.ShapeDtypeStruct((B, L, D), jnp.float32),
            jax.ShapeDtypeStruct((B, M, M), jnp.float32),
        ],
        scratch_shapes=[
            pltpu.VMEM((_C, _C), jnp.bfloat16),
            pltpu.VMEM((_C, M), jnp.float32),
            pltpu.VMEM((_C, M), jnp.float32),
        ],
        compiler_params=pltpu.CompilerParams(
            dimension_semantics=("arbitrary", "arbitrary"),
        ),
    )(scal, x, w_all, wo16, bo2)
    return y, s_final


# speedup vs baseline: 1.0910x; 1.0910x over previous
import functools

import jax
import jax.numpy as jnp
from jax import lax
from jax.experimental import pallas as pl
from jax.experimental.pallas import tpu as pltpu

_DECAY_MIN = 0.9
_DECAY_MAX = 0.999

_C = 256
_K = 4


def _bf(x):
    return x.astype(jnp.bfloat16)


def _body(scal_ref, x_ref, w_ref, wo_ref, bo_ref, y_ref, sfin_ref,
          mask_scr, dpow_scr, rpow_scr, *, M):
    c = pl.program_id(1)
    bg = scal_ref[0]
    ld = scal_ref[1]
    decay_c = scal_ref[2]

    @pl.when(c == 0)
    def _init():
        ti = lax.broadcasted_iota(jnp.int32, (_C, _C), 0).astype(jnp.float32)
        si = lax.broadcasted_iota(jnp.int32, (_C, _C), 1).astype(jnp.float32)
        mask_scr[...] = _bf(jnp.where(si <= ti, jnp.exp((ti - si) * ld), 0.0))
        tc = lax.broadcasted_iota(jnp.int32, (_C, M), 0).astype(jnp.float32)
        dpow_scr[...] = jnp.exp((tc + 1.0) * ld)
        rpow_scr[...] = jnp.exp((_C - 1.0 - tc) * ld)
        sfin_ref[...] = jnp.zeros_like(sfin_ref)

    mask = mask_scr[...]
    dpow = dpow_scr[...]
    rpow = rpow_scr[...]
    s = sfin_ref[0]

    for j in range(_K):
        xb = _bf(x_ref[0, j * _C:(j + 1) * _C, :])
        qkvg = jnp.dot(xb, w_ref[...], preferred_element_type=jnp.float32)
        q = qkvg[:, 0:M]
        k = qkvg[:, M:2 * M]
        v = qkvg[:, 2 * M:3 * M]
        g = jax.nn.sigmoid(qkvg[:, 3 * M:4 * M] + bg)
        gk = k * g
        gv16 = _bf(v * g)

        a = lax.dot_general(_bf(q), _bf(gk), (((1,), (1,)), ((), ())),
                            preferred_element_type=jnp.float32)
        a16 = _bf(a) * mask
        intra = jnp.dot(a16, gv16, preferred_element_type=jnp.float32)

        inter = jnp.dot(_bf(q * dpow), _bf(s),
                        preferred_element_type=jnp.float32)

        out = _bf(intra + inter)
        y_ref[0, j * _C:(j + 1) * _C, :] = (
            jnp.dot(out, wo_ref[...], preferred_element_type=jnp.float32)
            + bo_ref[...])

        ktv = lax.dot_general(_bf(gk * rpow), gv16, (((0,), (0,)), ((), ())),
                              preferred_element_type=jnp.float32)
        s = decay_c * s + ktv

    sfin_ref[0] = s


def kernel(x, Wq, Wk, Wv, Wo, bo, Wg, bg, decay_param):
    B, L, D = x.shape
    M = Wq.shape[1]
    nsteps = L // (_C * _K)

    decay = _DECAY_MIN + jax.nn.sigmoid(decay_param[0]) * (_DECAY_MAX - _DECAY_MIN)
    ld = jnp.log(decay)
    scal = jnp.stack([bg[0], ld, decay ** _C]).astype(jnp.float32)

    w_all = jnp.concatenate(
        [Wq, Wk, Wv, jnp.tile(Wg, (1, M))], axis=1).astype(jnp.bfloat16)
    wo16 = Wo.astype(jnp.bfloat16)
    bo2 = bo.reshape(1, D)

    body = functools.partial(_body, M=M)
    y, s_final = pl.pallas_call(
        body,
        grid=(B, nsteps),
        in_specs=[
            pl.BlockSpec(memory_space=pltpu.SMEM),
            pl.BlockSpec((1, _C * _K, D), lambda b, c: (b, c, 0)),
            pl.BlockSpec((D, 4 * M), lambda b, c: (0, 0)),
            pl.BlockSpec((M, D), lambda b, c: (0, 0)),
            pl.BlockSpec((1, D), lambda b, c: (0, 0)),
        ],
        out_specs=[
            pl.BlockSpec((1, _C * _K, D), lambda b, c: (b, c, 0)),
            pl.BlockSpec((1, M, M), lambda b, c: (b, 0, 0)),
        ],
        out_shape=[
            jax.ShapeDtypeStruct((B, L, D), jnp.float32),
            jax.ShapeDtypeStruct((B, M, M), jnp.float32),
        ],
        scratch_shapes=[
            pltpu.VMEM((_C, _C), jnp.bfloat16),
            pltpu.VMEM((_C, M), jnp.float32),
            pltpu.VMEM((_C, M), jnp.float32),
        ],
        compiler_params=pltpu.CompilerParams(
            dimension_semantics=("arbitrary", "arbitrary"),
        ),
    )(scal, x, w_all, wo16, bo2)
    return y, s_final
